# Initial kernel scaffold; baseline (speedup 1.0000x reference)
import jax
import jax.numpy as jnp
from jax.experimental import pallas as pl


def kernel(inputs, edge_index, W_self1, W_neigh1, b1, W_self2, W_neigh2, b2):
    raise NotImplementedError("write your pallas kernel here")


# speedup vs baseline: 7.2424x; 7.2424x over previous
import functools

import jax
import jax.numpy as jnp
from jax import lax
from jax.experimental import pallas as pl
from jax.experimental.pallas import tpu as pltpu
from jax.experimental.pallas import tpu_sc as plsc

N = 10000
E = 320000
IN_FEATS = 128
HIDDEN = 256

NC = 2
NS = 16
EPT = 10240
E_PAD = NC * NS * EPT
ROWS = 10176
RPT = ROWS // NS
D1 = 144
D2 = 16
CHUNK1 = 64
CHUNK2 = 128


def _make_seg_sum(D, CHUNK):
    d_lanes = D // 16
    nch = EPT // CHUNK
    mesh = plsc.VectorSubcoreMesh(core_axis_name="c", subcore_axis_name="s")

    @functools.partial(
        pl.kernel,
        mesh=mesh,
        compiler_params=pltpu.CompilerParams(use_tc_tiling_on_sc=False),
        out_type=jax.ShapeDtypeStruct((NC, NS, RPT, D), jnp.float32),
        scratch_types=[
            pltpu.VMEM((nch, CHUNK), jnp.int32),
            pltpu.VMEM((nch, CHUNK), jnp.int32),
            pltpu.VMEM((2, CHUNK, D), jnp.float32),
            pltpu.VMEM_SHARED((ROWS, D), jnp.float32),
            pltpu.SemaphoreType.DMA,
            pltpu.SemaphoreType.DMA,
        ],
    )
    def seg_sum(table_hbm, src_hbm, dst_hbm, out_hbm, srcv, dstv, rows, acc,
                sem0, sem1):
        c = lax.axis_index("c")
        s = lax.axis_index("s")
        pltpu.sync_copy(src_hbm.at[c, s], srcv)
        pltpu.sync_copy(dst_hbm.at[c, s], dstv)

        def zbody(t, carry):
            i = t // d_lanes
            j = t - i * d_lanes
            rows[0, i, pl.ds(j * 16, 16)] = jnp.zeros((16,), jnp.float32)
            return carry

        lax.fori_loop(0, CHUNK * d_lanes, zbody, 0)
        for z in range(RPT // CHUNK):
            pltpu.sync_copy(rows.at[0],
                            acc.at[pl.ds(s * RPT + z * CHUNK, CHUNK)])
        rem = RPT % CHUNK
        if rem:
            pltpu.sync_copy(
                rows.at[0, pl.ds(0, rem)],
                acc.at[pl.ds(s * RPT + (RPT // CHUNK) * CHUNK, rem)])

        pltpu.make_async_copy(table_hbm.at[srcv.at[0]], rows.at[0], sem0).start()
        pltpu.make_async_copy(table_hbm.at[srcv.at[1]], rows.at[1], sem1).start()
        plsc.subcore_barrier()

        def body(j, carry):
            i0 = 2 * j
            pltpu.make_async_copy(
                table_hbm.at[srcv.at[i0]], rows.at[0], sem0).wait()
            pltpu.sync_copy(rows.at[0], acc.at[dstv.at[i0]], add=True)

            @pl.when(j < nch // 2 - 1)
            def _():
                pltpu.make_async_copy(
                    table_hbm.at[srcv.at[i0 + 2]], rows.at[0], sem0).start()

            i1 = i0 + 1
            pltpu.make_async_copy(
                table_hbm.at[srcv.at[i1]], rows.at[1], sem1).wait()
            pltpu.sync_copy(rows.at[1], acc.at[dstv.at[i1]], add=True)

            @pl.when(j < nch // 2 - 1)
            def _():
                pltpu.make_async_copy(
                    table_hbm.at[srcv.at[i1 + 2]], rows.at[1], sem1).start()

            return carry

        lax.fori_loop(0, nch // 2, body, 0)
        plsc.subcore_barrier()
        pltpu.sync_copy(acc.at[pl.ds(s * RPT, RPT)], out_hbm.at[c, s])

    return seg_sum


_seg_sum_d1 = _make_seg_sum(D1, CHUNK1)
_seg_sum_d2 = _make_seg_sum(D2, CHUNK2)

_R = 2000


def _dense_body(x_ref, acc_ref, ws1_ref, wn1_ref, b1_ref, ws2_ref, wn2_ref,
                b2_ref, paug_ref, s_ref, rinv_ref):
    a = acc_ref[0] + acc_ref[1]
    deg = jnp.maximum(a[:, IN_FEATS:IN_FEATS + 1], 1.0)
    hn = a[:, :IN_FEATS] / deg
    h = jnp.dot(x_ref[...], ws1_ref[...], preferred_element_type=jnp.float32)
    h = h + jnp.dot(hn, wn1_ref[...], preferred_element_type=jnp.float32)
    h = jnp.maximum(h + b1_ref[...], 0.0)
    paug_ref[...] = jnp.dot(h, wn2_ref[...], preferred_element_type=jnp.float32)
    s_ref[...] = (jnp.dot(h, ws2_ref[...], preferred_element_type=jnp.float32)
                  + b2_ref[...])
    rinv_ref[...] = 1.0 / deg


def _epilogue_body(s_ref, acc2_ref, rinv_ref, out_ref):
    a2 = acc2_ref[0] + acc2_ref[1]
    out_ref[...] = s_ref[...] + a2 * rinv_ref[...]


def kernel(inputs, edge_index, W_self1, W_neigh1, b1, W_self2, W_neigh2, b2):
    x = inputs
    src = edge_index[0].astype(jnp.int32)
    dst = edge_index[1].astype(jnp.int32)
    pad = E_PAD - E
    src_p = jnp.concatenate([src, jnp.zeros((pad,), jnp.int32)])
    dst_p = jnp.concatenate([dst, jnp.full((pad,), N, jnp.int32)])
    src41 = src_p.reshape(NC, NS, EPT // CHUNK1, CHUNK1)
    dst41 = dst_p.reshape(NC, NS, EPT // CHUNK1, CHUNK1)
    src42 = src_p.reshape(NC, NS, EPT // CHUNK2, CHUNK2)
    dst42 = dst_p.reshape(NC, NS, EPT // CHUNK2, CHUNK2)

    xaug = jnp.concatenate(
        [x, jnp.ones((N, 1), x.dtype), jnp.zeros((N, D1 - IN_FEATS - 1),
                                                 x.dtype)], axis=1)
    acc1 = _seg_sum_d1(xaug, src41, dst41).reshape(NC, ROWS, D1)[:, :N]

    grid = (N // _R,)
    full = lambda shape: pl.BlockSpec(shape, lambda i: (0,) * len(shape))
    paug, s16, rinv = pl.pallas_call(
        _dense_body,
        grid=grid,
        in_specs=[
            pl.BlockSpec((_R, IN_FEATS), lambda i: (i, 0)),
            pl.BlockSpec((NC, _R, D1), lambda i: (0, i, 0)),
            full((IN_FEATS, HIDDEN)),
            full((IN_FEATS, HIDDEN)),
            full((1, HIDDEN)),
            full((HIDDEN, D2)),
            full((HIDDEN, D2)),
            full((1, D2)),
        ],
        out_specs=[
            pl.BlockSpec((_R, D2), lambda i: (i, 0)),
            pl.BlockSpec((_R, D2), lambda i: (i, 0)),
            pl.BlockSpec((_R, 1), lambda i: (i, 0)),
        ],
        out_shape=[
            jax.ShapeDtypeStruct((N, D2), jnp.float32),
            jax.ShapeDtypeStruct((N, D2), jnp.float32),
            jax.ShapeDtypeStruct((N, 1), jnp.float32),
        ],
    )(x, acc1, W_self1, W_neigh1, b1.reshape(1, HIDDEN),
      jnp.pad(W_self2, ((0, 0), (0, D2 - 2))),
      jnp.pad(W_neigh2, ((0, 0), (0, D2 - 2))),
      jnp.pad(b2, (0, D2 - 2)).reshape(1, D2))

    acc2 = _seg_sum_d2(paug, src42, dst42).reshape(NC, ROWS, D2)[:, :N]

    out16 = pl.pallas_call(
        _epilogue_body,
        grid=grid,
        in_specs=[
            pl.BlockSpec((_R, D2), lambda i: (i, 0)),
            pl.BlockSpec((NC, _R, D2), lambda i: (0, i, 0)),
            pl.BlockSpec((_R, 1), lambda i: (i, 0)),
        ],
        out_specs=pl.BlockSpec((_R, D2), lambda i: (i, 0)),
        out_shape=jax.ShapeDtypeStruct((N, D2), jnp.float32),
    )(s16, acc2, rinv)
    return out16[:, :2]


# speedup vs baseline: 7.2457x; 1.0005x over previous
import functools

import jax
import jax.numpy as jnp
from jax import lax
from jax.experimental import pallas as pl
from jax.experimental.pallas import tpu as pltpu
from jax.experimental.pallas import tpu_sc as plsc

N = 10000
E = 320000
IN_FEATS = 128
HIDDEN = 256

NC = 2
NS = 16
EPT = 10240
E_PAD = NC * NS * EPT
ROWS = 10176
RPT = ROWS // NS
D1 = 144
D2 = 16
CHUNK1 = 64
CHUNK2 = 128
A1, B1 = 160, 160
A2, B2 = 80, 80
PKMAX1 = 240
PKMAX2 = 120


def _make_seg_sum(D, CHUNK, A, B, PKMAX):
    d_lanes = D // 16
    mesh = plsc.VectorSubcoreMesh(core_axis_name="c", subcore_axis_name="s")

    @functools.partial(
        pl.kernel,
        mesh=mesh,
        compiler_params=pltpu.CompilerParams(use_tc_tiling_on_sc=False),
        out_type=jax.ShapeDtypeStruct((NC, NS, RPT, D), jnp.float32),
        scratch_types=[
            pltpu.VMEM((PKMAX, CHUNK), jnp.int32),
            pltpu.VMEM((2, CHUNK), jnp.int32),
            pltpu.VMEM((2, CHUNK), jnp.int32),
            pltpu.VMEM((2, CHUNK, D), jnp.float32),
            pltpu.VMEM_SHARED((ROWS, D), jnp.float32),
            pltpu.SemaphoreType.DMA,
            pltpu.SemaphoreType.DMA,
        ],
    )
    def seg_sum(table_hbm, pk_hbm, out_hbm, pk, srcb, dstb, rows, acc,
                sem0, sem1):
        c = lax.axis_index("c")
        s = lax.axis_index("s")
        sems = (sem0, sem1)

        def zbody(t, carry):
            i = t // d_lanes
            j = t - i * d_lanes
            rows[0, i, pl.ds(j * 16, 16)] = jnp.zeros((16,), jnp.float32)
            return carry

        lax.fori_loop(0, CHUNK * d_lanes, zbody, 0)
        for z in range(RPT // CHUNK):
            pltpu.sync_copy(rows.at[0],
                            acc.at[pl.ds(s * RPT + z * CHUNK, CHUNK)])
        rem = RPT % CHUNK
        if rem:
            pltpu.sync_copy(
                rows.at[0, pl.ds(0, rem)],
                acc.at[pl.ds(s * RPT + (RPT // CHUNK) * CHUNK, rem)])

        def unpack(i, slot):
            for t in range(CHUNK // 16):
                v = pk[i, pl.ds(t * 16, 16)]
                srcb[slot, pl.ds(t * 16, 16)] = v & 16383
                dstb[slot, pl.ds(t * 16, 16)] = lax.shift_right_logical(v, 14)

        def gather_start(slot):
            pltpu.make_async_copy(
                table_hbm.at[srcb.at[slot]], rows.at[slot], sems[slot]).start()

        def gather_wait(slot):
            pltpu.make_async_copy(
                table_hbm.at[srcb.at[slot]], rows.at[slot], sems[slot]).wait()

        def run(nch, base):
            pltpu.sync_copy(pk_hbm.at[pl.ds(base, nch)],
                            pk.at[pl.ds(0, nch)])
            unpack(0, 0)
            gather_start(0)
            unpack(1, 1)
            gather_start(1)
            plsc.subcore_barrier()

            def body(j, carry):
                for slot in range(2):
                    i = 2 * j + slot
                    gather_wait(slot)
                    pltpu.sync_copy(rows.at[slot], acc.at[dstb.at[slot]],
                                    add=True)

                    @pl.when(j < nch // 2 - 1)
                    def _():
                        unpack(i + 2, slot)
                        gather_start(slot)

                return carry

            lax.fori_loop(0, nch // 2, body, 0)

        @pl.when(c == 0)
        def _():
            run(A, s * A)

        @pl.when(c == 1)
        def _():
            run(B, NS * A + s * B)

        plsc.subcore_barrier()
        pltpu.sync_copy(acc.at[pl.ds(s * RPT, RPT)], out_hbm.at[c, s])

    return seg_sum


_seg_sum_d1 = _make_seg_sum(D1, CHUNK1, A1, B1, PKMAX1)
_seg_sum_d2 = _make_seg_sum(D2, CHUNK2, A2, B2, PKMAX2)

_R = 2000


def _dense_body(x_ref, acc_ref, ws1_ref, wn1_ref, b1_ref, ws2_ref, wn2_ref,
                b2_ref, paug_ref, s_ref, rinv_ref):
    a = acc_ref[0] + acc_ref[1]
    deg = jnp.maximum(a[:, IN_FEATS:IN_FEATS + 1], 1.0)
    hn = a[:, :IN_FEATS] / deg
    h = jnp.dot(x_ref[...], ws1_ref[...], preferred_element_type=jnp.float32)
    h = h + jnp.dot(hn, wn1_ref[...], preferred_element_type=jnp.float32)
    h = jnp.maximum(h + b1_ref[...], 0.0)
    paug_ref[...] = jnp.dot(h, wn2_ref[...], preferred_element_type=jnp.float32)
    s_ref[...] = (jnp.dot(h, ws2_ref[...], preferred_element_type=jnp.float32)
                  + b2_ref[...])
    rinv_ref[...] = 1.0 / deg


def _epilogue_body(s_ref, acc2_ref, rinv_ref, out_ref):
    a2 = acc2_ref[0] + acc2_ref[1]
    out_ref[...] = (s_ref[...] + a2 * rinv_ref[...])[:, :2]


def kernel(inputs, edge_index, W_self1, W_neigh1, b1, W_self2, W_neigh2, b2):
    x = inputs
    src = edge_index[0].astype(jnp.int32)
    dst = edge_index[1].astype(jnp.int32)
    pad = E_PAD - E
    src_p = jnp.concatenate([src, jnp.zeros((pad,), jnp.int32)])
    dst_p = jnp.concatenate([dst, jnp.full((pad,), N, jnp.int32)])
    packed = src_p | (dst_p << 14)
    pk1 = packed.reshape(E_PAD // CHUNK1, CHUNK1)
    pk2 = packed.reshape(E_PAD // CHUNK2, CHUNK2)

    xaug = jnp.concatenate(
        [x, jnp.ones((N, 1), x.dtype), jnp.zeros((N, D1 - IN_FEATS - 1),
                                                 x.dtype)], axis=1)
    acc1 = _seg_sum_d1(xaug, pk1).reshape(NC, ROWS, D1)[:, :N]

    grid = (N // _R,)
    full = lambda shape: pl.BlockSpec(shape, lambda i: (0,) * len(shape))
    paug, s16, rinv = pl.pallas_call(
        _dense_body,
        grid=grid,
        in_specs=[
            pl.BlockSpec((_R, IN_FEATS), lambda i: (i, 0)),
            pl.BlockSpec((NC, _R, D1), lambda i: (0, i, 0)),
            full((IN_FEATS, HIDDEN)),
            full((IN_FEATS, HIDDEN)),
            full((1, HIDDEN)),
            full((HIDDEN, D2)),
            full((HIDDEN, D2)),
            full((1, D2)),
        ],
        out_specs=[
            pl.BlockSpec((_R, D2), lambda i: (i, 0)),
            pl.BlockSpec((_R, D2), lambda i: (i, 0)),
            pl.BlockSpec((_R, 1), lambda i: (i, 0)),
        ],
        out_shape=[
            jax.ShapeDtypeStruct((N, D2), jnp.float32),
            jax.ShapeDtypeStruct((N, D2), jnp.float32),
            jax.ShapeDtypeStruct((N, 1), jnp.float32),
        ],
    )(x, acc1, W_self1, W_neigh1, b1.reshape(1, HIDDEN),
      jnp.pad(W_self2, ((0, 0), (0, D2 - 2))),
      jnp.pad(W_neigh2, ((0, 0), (0, D2 - 2))),
      jnp.pad(b2, (0, D2 - 2)).reshape(1, D2))

    acc2 = _seg_sum_d2(paug, pk2).reshape(NC, ROWS, D2)[:, :N]

    out = pl.pallas_call(
        _epilogue_body,
        grid=grid,
        in_specs=[
            pl.BlockSpec((_R, D2), lambda i: (i, 0)),
            pl.BlockSpec((NC, _R, D2), lambda i: (0, i, 0)),
            pl.BlockSpec((_R, 1), lambda i: (i, 0)),
        ],
        out_specs=pl.BlockSpec((_R, 2), lambda i: (i, 0)),
        out_shape=jax.ShapeDtypeStruct((N, 2), jnp.float32),
    )(s16, acc2, rinv)
    return out
